# Initial kernel scaffold; baseline (speedup 1.0000x reference)
import jax
import jax.numpy as jnp
from jax.experimental import pallas as pl


def kernel(item_features, edge_index, edge_weight, W0, W1, sess_items):
    raise NotImplementedError("write your pallas kernel here")


# speedup vs baseline: 4.0301x; 4.0301x over previous
import functools

import jax
import jax.numpy as jnp
from jax import lax
from jax.experimental import pallas as pl
from jax.experimental.pallas import tpu as pltpu
from jax.experimental.pallas import tpu_sc as plsc

N = 10000
E = 320000
D = 128
H = 64
B = 4096
L = 50

NC = 2
NS = 16
G = 128

EHALF = E // NC
EPT_P = 10240
ESC_P = EPT_P * NS
PAD_P = ESC_P - EHALF
NP = 10240

BL = B * L
SHALF = BL // NC
EPT_S = 7168
ESC_S = EPT_S * NS
PAD_S = ESC_S - SHALF
BC = B // NC


def _build_scatter(acc_rows, z_rows, o_rows, ept, cchunk, with_w,
                   out_scale, out_core_stride):
    esc = ept * NS
    escg = esc // G
    eptg = ept // G
    nch = ept // cchunk
    k = cchunk // G

    def impl(x_hbm, src_hbm, dst_hbm, w_hbm, zeros_hbm, out_hbm,
             acc, sidx, didx, w_v, rows, sem):
        c = lax.axis_index("c")
        s = lax.axis_index("s")
        pltpu.sync_copy(zeros_hbm, acc.at[pl.ds(s * z_rows, z_rows)])
        plsc.subcore_barrier()

        def chunk_body(ci, carry):
            gs = c * escg + s * eptg + ci * k
            pltpu.sync_copy(src_hbm.at[pl.ds(gs, k)], sidx)
            pltpu.sync_copy(dst_hbm.at[pl.ds(gs, k)], didx)
            if with_w:
                pltpu.sync_copy(
                    w_hbm.at[pl.ds(c * esc + s * ept + ci * cchunk, cchunk)],
                    w_v)
            cps = [pltpu.async_copy(x_hbm.at[sidx.at[j]],
                                    rows.at[pl.ds(j * G, G)], sem)
                   for j in range(k)]
            for cp in cps:
                cp.wait()
            if with_w:
                def scale_body(g16, cc):
                    base = g16 * 16
                    w16 = w_v[pl.ds(base, 16)]
                    for j in range(16):
                        wk = w16[j]
                        for q in range(H // 16):
                            v = rows[base + j, pl.ds(16 * q, 16)]
                            rows[base + j, pl.ds(16 * q, 16)] = v * wk
                    return cc
                lax.fori_loop(0, cchunk // 16, scale_body, 0)
            for j in range(k):
                pltpu.sync_copy(rows.at[pl.ds(j * G, G)],
                                acc.at[didx.at[j]], add=True)
            return carry

        lax.fori_loop(0, nch, chunk_body, 0)
        plsc.subcore_barrier()

        obase = c * out_core_stride + s * o_rows
        if out_scale == 1.0:
            pltpu.sync_copy(acc.at[pl.ds(s * o_rows, o_rows)],
                            out_hbm.at[pl.ds(obase, o_rows)])
        else:
            pltpu.sync_copy(acc.at[pl.ds(s * o_rows, o_rows)],
                            rows.at[pl.ds(0, o_rows)])
            def out_body(r, cc):
                for q in range(H // 16):
                    v = rows[r, pl.ds(16 * q, 16)]
                    rows[r, pl.ds(16 * q, 16)] = v * out_scale
                return cc
            lax.fori_loop(0, o_rows, out_body, 0)
            pltpu.sync_copy(rows.at[pl.ds(0, o_rows)],
                            out_hbm.at[pl.ds(obase, o_rows)])

    scratch = [
        pltpu.VMEM_SHARED((acc_rows, H), jnp.float32),
        pltpu.VMEM((k, G), jnp.int32),
        pltpu.VMEM((k, G), jnp.int32),
        pltpu.VMEM((cchunk,), jnp.float32),
        pltpu.VMEM((cchunk, H), jnp.float32),
        pltpu.SemaphoreType.DMA,
    ]
    mesh = plsc.VectorSubcoreMesh(core_axis_name="c", subcore_axis_name="s")
    n_out = out_core_stride * NC
    out_type = jax.ShapeDtypeStruct((n_out, H), jnp.float32)
    cparams = pltpu.CompilerParams(use_tc_tiling_on_sc=False)

    if with_w:
        def body(x, src, dst, w, z, out, acc, sidx, didx, w_v, rows, sem):
            impl(x, src, dst, w, z, out, acc, sidx, didx, w_v, rows, sem)
        return pl.kernel(body, out_type=out_type, mesh=mesh,
                         scratch_types=scratch, compiler_params=cparams)
    else:
        def body(x, src, dst, z, out, acc, sidx, didx, rows, sem):
            impl(x, src, dst, None, z, out, acc, sidx, didx, None, rows, sem)
        return pl.kernel(body, out_type=out_type, mesh=mesh,
                         scratch_types=scratch[:3] + scratch[4:],
                         compiler_params=cparams)


_prop = _build_scatter(acc_rows=NP, z_rows=NP // NS, o_rows=NP // NS,
                       ept=EPT_P, cchunk=1024, with_w=True, out_scale=1.0,
                       out_core_stride=NP)
_pool = _build_scatter(acc_rows=BC + 16, z_rows=BC // NS,
                       o_rows=BC // NS, ept=EPT_S, cchunk=1024,
                       with_w=False, out_scale=1.0 / L,
                       out_core_stride=BC)

_BR = 1000


def _mm1_body(x_ref, w_ref, o_ref):
    o_ref[...] = jnp.dot(x_ref[...], w_ref[...],
                         preferred_element_type=jnp.float32)


_mm1 = pl.pallas_call(
    _mm1_body,
    grid=(N // _BR,),
    in_specs=[pl.BlockSpec((_BR, D), lambda i: (i, 0)),
              pl.BlockSpec((D, H), lambda i: (0, 0))],
    out_specs=pl.BlockSpec((_BR, H), lambda i: (i, 0)),
    out_shape=jax.ShapeDtypeStruct((N, H), jnp.float32),
)


def _mm2_body(p_ref, w_ref, o_ref):
    h = p_ref[0] + p_ref[1]
    o_ref[...] = jnp.dot(h, w_ref[...], preferred_element_type=jnp.float32)


_mm2 = pl.pallas_call(
    _mm2_body,
    grid=(N // _BR,),
    in_specs=[pl.BlockSpec((2, _BR, H), lambda i: (0, i, 0)),
              pl.BlockSpec((H, H), lambda i: (0, 0))],
    out_specs=pl.BlockSpec((_BR, H), lambda i: (i, 0)),
    out_shape=jax.ShapeDtypeStruct((N, H), jnp.float32),
)


def _add_body(p_ref, o_ref):
    o_ref[...] = p_ref[0] + p_ref[1]


_madd = pl.pallas_call(
    _add_body,
    grid=(N // _BR,),
    in_specs=[pl.BlockSpec((2, _BR, H), lambda i: (0, i, 0))],
    out_specs=pl.BlockSpec((_BR, H), lambda i: (i, 0)),
    out_shape=jax.ShapeDtypeStruct((N, H), jnp.float32),
)


def _pad_half(a, pad, value):
    return jnp.concatenate([a, jnp.full((pad,), value, a.dtype)])


def kernel(item_features, edge_index, edge_weight, W0, W1, sess_items):
    src = edge_index[0]
    dst = edge_index[1]
    src2 = jnp.concatenate([_pad_half(src[:EHALF], PAD_P, 0),
                            _pad_half(src[EHALF:], PAD_P, 0)]).reshape(-1, G)
    dst2 = jnp.concatenate([_pad_half(dst[:EHALF], PAD_P, N),
                            _pad_half(dst[EHALF:], PAD_P, N)]).reshape(-1, G)
    w_p = jnp.concatenate([_pad_half(edge_weight[:EHALF], PAD_P, 0.0),
                           _pad_half(edge_weight[EHALF:], PAD_P, 0.0)])

    sess = sess_items.reshape(-1)
    sess2 = jnp.concatenate([_pad_half(sess[:SHALF], PAD_S, 0),
                             _pad_half(sess[SHALF:], PAD_S, 0)]).reshape(-1, G)
    sdst = jnp.repeat(jnp.arange(B, dtype=jnp.int32) % BC, L)
    sdst2 = jnp.concatenate([_pad_half(sdst[:SHALF], PAD_S, BC),
                             _pad_half(sdst[SHALF:], PAD_S, BC)]
                            ).reshape(-1, G)

    zp = jnp.zeros((NP // NS, H), jnp.float32)
    zs = jnp.zeros((BC // NS, H), jnp.float32)

    g0 = _mm1(item_features, W0)
    p1 = _prop(g0, src2, dst2, w_p, zp)
    g1 = _mm2(p1.reshape(2, NP, H), W1)
    p2 = _prop(g1, src2, dst2, w_p, zp)
    h2 = _madd(p2.reshape(2, NP, H))
    s_emb = _pool(h2, sess2, sdst2, zs)
    return (s_emb, h2)


# speedup vs baseline: 4.3045x; 1.0681x over previous
import functools

import jax
import jax.numpy as jnp
from jax import lax
from jax.experimental import pallas as pl
from jax.experimental.pallas import tpu as pltpu
from jax.experimental.pallas import tpu_sc as plsc

N = 10000
E = 320000
D = 128
H = 64
B = 4096
L = 50

NC = 2
NS = 16
G = 128

EHALF = E // NC
EPT_P = 10240
ESC_P = EPT_P * NS
PAD_P = ESC_P - EHALF
NP = 10240

BL = B * L
SHALF = BL // NC
EPT_S = 7168
ESC_S = EPT_S * NS
PAD_S = ESC_S - SHALF
BC = B // NC


def _build_scatter(acc_rows, z_rows, o_rows, ept, cchunk, with_w,
                   out_core_stride):
    esc = ept * NS
    escg = esc // G
    eptg = ept // G
    nch = ept // cchunk
    k = cchunk // G
    assert nch % 2 == 0

    def impl(x_hbm, src_hbm, dst_hbm, w_hbm, zeros_hbm, out_hbm,
             acc, sidx, didx, w_v, rows, sems):
        c = lax.axis_index("c")
        s = lax.axis_index("s")
        pltpu.sync_copy(zeros_hbm, acc.at[pl.ds(s * z_rows, z_rows)])
        plsc.subcore_barrier()

        def load_idx(ci, b):
            gs = c * escg + s * eptg + ci * k
            pltpu.sync_copy(src_hbm.at[pl.ds(gs, k)], sidx[b])
            pltpu.sync_copy(dst_hbm.at[pl.ds(gs, k)], didx[b])
            if with_w:
                pltpu.sync_copy(
                    w_hbm.at[pl.ds(c * esc + s * ept + ci * cchunk, cchunk)],
                    w_v[b])

        def fire(b):
            for j in range(k):
                pltpu.async_copy(x_hbm.at[sidx[b].at[j]],
                                 rows[b].at[pl.ds(j * G, G)], sems[b])

        def drain(b):
            for j in range(k):
                pltpu.make_async_copy(x_hbm.at[sidx[b].at[j]],
                                      rows[b].at[pl.ds(j * G, G)],
                                      sems[b]).wait()

        def process(b):
            if with_w:
                def scale_body(g16, cc):
                    base = g16 * 16
                    w16 = w_v[b][pl.ds(base, 16)]
                    for j in range(16):
                        wk = w16[j]
                        for q in range(H // 16):
                            v = rows[b][base + j, pl.ds(16 * q, 16)]
                            rows[b][base + j, pl.ds(16 * q, 16)] = v * wk
                    return cc
                lax.fori_loop(0, cchunk // 16, scale_body, 0)
            for j in range(k):
                pltpu.sync_copy(rows[b].at[pl.ds(j * G, G)],
                                acc.at[didx[b].at[j]], add=True)

        load_idx(0, 0)
        fire(0)

        def chunk_body(i2, carry):
            ci = i2 * 2
            load_idx(ci + 1, 1)
            fire(1)
            drain(0)
            process(0)

            @pl.when(ci + 2 < nch)
            def _():
                load_idx(ci + 2, 0)
                fire(0)
            drain(1)
            process(1)
            return carry

        lax.fori_loop(0, nch // 2, chunk_body, 0)
        plsc.subcore_barrier()

        obase = c * out_core_stride + s * o_rows
        pltpu.sync_copy(acc.at[pl.ds(s * o_rows, o_rows)],
                        out_hbm.at[pl.ds(obase, o_rows)])

    scratch = [
        pltpu.VMEM_SHARED((acc_rows, H), jnp.float32),
        [pltpu.VMEM((k, G), jnp.int32)] * 2,
        [pltpu.VMEM((k, G), jnp.int32)] * 2,
        [pltpu.VMEM((cchunk,), jnp.float32)] * 2,
        [pltpu.VMEM((cchunk, H), jnp.float32)] * 2,
        [pltpu.SemaphoreType.DMA] * 2,
    ]
    mesh = plsc.VectorSubcoreMesh(core_axis_name="c", subcore_axis_name="s")
    n_out = out_core_stride * NC
    out_type = jax.ShapeDtypeStruct((n_out, H), jnp.float32)
    cparams = pltpu.CompilerParams(use_tc_tiling_on_sc=False)

    if with_w:
        def body(x, src, dst, w, z, out, acc, sidx, didx, w_v, rows, sem):
            impl(x, src, dst, w, z, out, acc, sidx, didx, w_v, rows, sem)
        return pl.kernel(body, out_type=out_type, mesh=mesh,
                         scratch_types=scratch, compiler_params=cparams)
    else:
        def body(x, src, dst, z, out, acc, sidx, didx, rows, sem):
            impl(x, src, dst, None, z, out, acc, sidx, didx, None, rows, sem)
        return pl.kernel(body, out_type=out_type, mesh=mesh,
                         scratch_types=scratch[:3] + scratch[4:],
                         compiler_params=cparams)


_prop = _build_scatter(acc_rows=NP, z_rows=NP // NS, o_rows=NP // NS,
                       ept=EPT_P, cchunk=512, with_w=True,
                       out_core_stride=NP)
_pool = _build_scatter(acc_rows=B + 2 * G, z_rows=B // NS, o_rows=B // NS,
                       ept=EPT_S, cchunk=512, with_w=False,
                       out_core_stride=B)

_BR = 1000


def _mm1_body(x_ref, w_ref, o_ref):
    o_ref[...] = jnp.dot(x_ref[...], w_ref[...],
                         preferred_element_type=jnp.float32)


_mm1 = pl.pallas_call(
    _mm1_body,
    grid=(N // _BR,),
    in_specs=[pl.BlockSpec((_BR, D), lambda i: (i, 0)),
              pl.BlockSpec((D, H), lambda i: (0, 0))],
    out_specs=pl.BlockSpec((_BR, H), lambda i: (i, 0)),
    out_shape=jax.ShapeDtypeStruct((N, H), jnp.float32),
)


def _mm2_body(p_ref, w_ref, o_ref):
    h = p_ref[0] + p_ref[1]
    o_ref[...] = jnp.dot(h, w_ref[...], preferred_element_type=jnp.float32)


_mm2 = pl.pallas_call(
    _mm2_body,
    grid=(N // _BR,),
    in_specs=[pl.BlockSpec((2, _BR, H), lambda i: (0, i, 0)),
              pl.BlockSpec((H, H), lambda i: (0, 0))],
    out_specs=pl.BlockSpec((_BR, H), lambda i: (i, 0)),
    out_shape=jax.ShapeDtypeStruct((N, H), jnp.float32),
)


def _add_body(p_ref, o_ref):
    o_ref[...] = p_ref[0] + p_ref[1]


_madd = pl.pallas_call(
    _add_body,
    grid=(N // _BR,),
    in_specs=[pl.BlockSpec((2, _BR, H), lambda i: (0, i, 0))],
    out_specs=pl.BlockSpec((_BR, H), lambda i: (i, 0)),
    out_shape=jax.ShapeDtypeStruct((N, H), jnp.float32),
)


def _mean_body(p_ref, o_ref):
    o_ref[...] = (p_ref[0] + p_ref[1]) * (1.0 / L)


_BRS = 1024

_mean = pl.pallas_call(
    _mean_body,
    grid=(B // _BRS,),
    in_specs=[pl.BlockSpec((2, _BRS, H), lambda i: (0, i, 0))],
    out_specs=pl.BlockSpec((_BRS, H), lambda i: (i, 0)),
    out_shape=jax.ShapeDtypeStruct((B, H), jnp.float32),
)


def _pad_half(a, pad, value):
    return jnp.concatenate([a, jnp.full((pad,), value, a.dtype)])


def kernel(item_features, edge_index, edge_weight, W0, W1, sess_items):
    src = edge_index[0]
    dst = edge_index[1]
    src2 = jnp.concatenate([_pad_half(src[:EHALF], PAD_P, 0),
                            _pad_half(src[EHALF:], PAD_P, 0)]).reshape(-1, G)
    dst2 = jnp.concatenate([_pad_half(dst[:EHALF], PAD_P, N),
                            _pad_half(dst[EHALF:], PAD_P, N)]).reshape(-1, G)
    w_p = jnp.concatenate([_pad_half(edge_weight[:EHALF], PAD_P, 0.0),
                           _pad_half(edge_weight[EHALF:], PAD_P, 0.0)])

    sess = sess_items.T.reshape(-1)
    sess2 = jnp.concatenate([_pad_half(sess[:SHALF], PAD_S, 0),
                             _pad_half(sess[SHALF:], PAD_S, 0)]).reshape(-1, G)
    sdst = jnp.tile(jnp.arange(B, dtype=jnp.int32), L)
    sdst2 = jnp.concatenate([_pad_half(sdst[:SHALF], PAD_S, B),
                             _pad_half(sdst[SHALF:], PAD_S, B)]
                            ).reshape(-1, G)

    zp = jnp.zeros((NP // NS, H), jnp.float32)
    zs = jnp.zeros((B // NS, H), jnp.float32)

    g0 = _mm1(item_features, W0)
    p1 = _prop(g0, src2, dst2, w_p, zp)
    g1 = _mm2(p1.reshape(2, NP, H), W1)
    p2 = _prop(g1, src2, dst2, w_p, zp)
    h2 = _madd(p2.reshape(2, NP, H))
    ps = _pool(h2, sess2, sdst2, zs)
    s_emb = _mean(ps.reshape(2, B, H))
    return (s_emb, h2)
